# Initial kernel scaffold; baseline (speedup 1.0000x reference)
import jax
import jax.numpy as jnp
from jax.experimental import pallas as pl


def kernel(x, edge_index, W1, b1, W2, b2, W3, b3, Wc, bc):
    raise NotImplementedError("write your pallas kernel here")


# speedup vs baseline: 88.3113x; 88.3113x over previous
import functools

import jax
import jax.numpy as jnp
from jax import lax
from jax.experimental import pallas as pl
from jax.experimental.pallas import tpu as pltpu
from jax.experimental.pallas import tpu_sc as plsc

_N = 100000
_E = 6400000
_NC = 2
_NS = 16
_NW = _NC * _NS
_D8 = 8

CHUNK = 128
GRP = 8
CPT = 1568
GROUPS = CPT // GRP
E_PAD = _NW * CPT * CHUNK
NPAD = 100352
RPT = NPAD // _NS

_mesh = plsc.VectorSubcoreMesh(
    core_axis_name="c", subcore_axis_name="s", num_cores=_NC, num_subcores=_NS
)
_cparams = pltpu.CompilerParams(use_tc_tiling_on_sc=False)


def _make_deg():
    @functools.partial(
        pl.kernel,
        out_type=jax.ShapeDtypeStruct((_NC, NPAD), jnp.float32),
        mesh=_mesh,
        compiler_params=_cparams,
        scratch_types=[
            pltpu.VMEM_SHARED((NPAD,), jnp.float32),
            pltpu.VMEM((CHUNK,), jnp.float32),
            [pltpu.VMEM((CHUNK,), jnp.int32) for _ in range(GRP)],
            pltpu.SemaphoreType.DMA,
            pltpu.SemaphoreType.DMA,
        ],
    )
    def deg_kernel(dst_hbm, zz_hbm, ones_hbm, out_hbm, z_sp, ones_v, idxd, sem_i, sem_s):
        c = lax.axis_index("c")
        s = lax.axis_index("s")
        r0 = s * RPT
        pltpu.sync_copy(zz_hbm.at[pl.ds(r0, RPT)], z_sp.at[pl.ds(r0, RPT)])
        pltpu.sync_copy(ones_hbm, ones_v)
        plsc.subcore_barrier()
        base = (c * _NS + s) * CPT * CHUNK

        def group(g, carry):
            e0 = base + g * (GRP * CHUNK)
            ic = [
                pltpu.async_copy(
                    dst_hbm.at[pl.ds(e0 + j * CHUNK, CHUNK)], idxd[j], sem_i
                )
                for j in range(GRP)
            ]
            for d in ic:
                d.wait()
            ss = [
                pltpu.async_copy(ones_v, z_sp.at[idxd[j]], sem_s, add=True)
                for j in range(GRP)
            ]
            for d in ss:
                d.wait()
            return carry

        lax.fori_loop(0, GROUPS, group, 0)
        plsc.subcore_barrier()
        pltpu.sync_copy(z_sp.at[pl.ds(r0, RPT)], out_hbm.at[c, pl.ds(r0, RPT)])

    return deg_kernel


def _make_prop(D):
    one_d = D == 1
    tshape = (NPAD,) if one_d else (NPAD, D)
    rshape = (CHUNK,) if one_d else (CHUNK, D)

    @functools.partial(
        pl.kernel,
        out_type=jax.ShapeDtypeStruct((_NC,) + tshape, jnp.float32),
        mesh=_mesh,
        compiler_params=_cparams,
        scratch_types=[
            pltpu.VMEM_SHARED(tshape, jnp.float32),
            pltpu.VMEM_SHARED(tshape, jnp.float32),
            pltpu.VMEM((GRP * CHUNK,), jnp.int32),
            [pltpu.VMEM((CHUNK,), jnp.int32) for _ in range(GRP)],
            [pltpu.VMEM(rshape, jnp.float32) for _ in range(GRP)],
            pltpu.SemaphoreType.DMA,
            pltpu.SemaphoreType.DMA,
            pltpu.SemaphoreType.DMA,
        ],
    )
    def prop_kernel(
        src_hbm, dst_hbm, y_hbm, zz_hbm, out_hbm,
        y_sp, z_sp, idxs, idxd, rows, sem_i, sem_g, sem_s,
    ):
        c = lax.axis_index("c")
        s = lax.axis_index("s")
        r0 = s * RPT
        pltpu.sync_copy(y_hbm.at[pl.ds(r0, RPT)], y_sp.at[pl.ds(r0, RPT)])
        pltpu.sync_copy(zz_hbm.at[pl.ds(r0, RPT)], z_sp.at[pl.ds(r0, RPT)])
        plsc.subcore_barrier()
        base = (c * _NS + s) * CPT * CHUNK

        def group(g, carry):
            e0 = base + g * (GRP * CHUNK)
            ic = [
                pltpu.async_copy(
                    dst_hbm.at[pl.ds(e0 + j * CHUNK, CHUNK)], idxd[j], sem_i
                )
                for j in range(GRP)
            ]
            ic.append(
                pltpu.async_copy(src_hbm.at[pl.ds(e0, GRP * CHUNK)], idxs, sem_i)
            )
            for d in ic:
                d.wait()
            gs = [
                pltpu.async_copy(
                    y_sp.at[idxs.at[pl.ds(j * CHUNK, CHUNK)]], rows[j], sem_g
                )
                for j in range(GRP)
            ]
            for d in gs:
                d.wait()
            ss = [
                pltpu.async_copy(rows[j], z_sp.at[idxd[j]], sem_s, add=True)
                for j in range(GRP)
            ]
            for d in ss:
                d.wait()
            return carry

        lax.fori_loop(0, GROUPS, group, 0)
        plsc.subcore_barrier()
        pltpu.sync_copy(z_sp.at[pl.ds(r0, RPT)], out_hbm.at[c, pl.ds(r0, RPT)])

    return prop_kernel


_deg = _make_deg()
_prop1 = _make_prop(1)
_prop8 = _make_prop(_D8)


_BLK = 2048
_GRID = NPAD // _BLK


def _vspec(c):
    return pl.BlockSpec((_BLK, c), lambda i: (i, 0))


def _pspec(c):
    return pl.BlockSpec((_NC, _BLK, c), lambda i: (0, i, 0))


def _wspec(shape):
    return pl.BlockSpec(shape, lambda i: tuple(0 for _ in shape))


def _mm(a, w):
    k = w.shape[0]
    acc = a[:, 0:1] * w[0:1, :]
    for i in range(1, k):
        acc = acc + a[:, i : i + 1] * w[i : i + 1, :]
    return acc


def _pad8(a):
    return jnp.pad(a, ((0, 0), (0, _D8 - a.shape[1])))


def _stage_dis_body(degp_ref, x_ref, dis_ref, y1_ref):
    deg = degp_ref[0] + degp_ref[1] + 1.0
    dis = lax.rsqrt(deg)
    dis_ref[...] = dis
    y1_ref[...] = dis * x_ref[...]


_stage_dis = pl.pallas_call(
    _stage_dis_body,
    grid=(_GRID,),
    in_specs=[_pspec(1), _vspec(1)],
    out_specs=[_vspec(1), _vspec(1)],
    out_shape=[
        jax.ShapeDtypeStruct((NPAD, 1), jnp.float32),
        jax.ShapeDtypeStruct((NPAD, 1), jnp.float32),
    ],
)


def _stage1_body(dis_ref, zp_ref, y1_ref, w1_ref, b1_ref, y2_ref):
    dis = dis_ref[...]
    t = dis * (zp_ref[0] + zp_ref[1] + y1_ref[...])
    h1 = jnp.tanh(t * w1_ref[...] + b1_ref[...])
    y2_ref[...] = _pad8(dis * h1)


_stage1 = pl.pallas_call(
    _stage1_body,
    grid=(_GRID,),
    in_specs=[_vspec(1), _pspec(1), _vspec(1), _wspec((1, 4)), _wspec((1, 4))],
    out_specs=[_vspec(_D8)],
    out_shape=[jax.ShapeDtypeStruct((NPAD, _D8), jnp.float32)],
)


def _stage2_body(dis_ref, zp_ref, y2_ref, w2_ref, b2_ref, w3_ref, y3_ref):
    dis = dis_ref[...]
    u = dis * (zp_ref[0][:, :4] + zp_ref[1][:, :4] + y2_ref[:, :4])
    h2 = jnp.tanh(_mm(u, w2_ref[...]) + b2_ref[...])
    y3_ref[...] = _pad8(dis * _mm(h2, w3_ref[...]))


_stage2 = pl.pallas_call(
    _stage2_body,
    grid=(_GRID,),
    in_specs=[
        _vspec(1), _pspec(_D8), _vspec(_D8),
        _wspec((4, 4)), _wspec((1, 4)), _wspec((4, 2)),
    ],
    out_specs=[_vspec(_D8)],
    out_shape=[jax.ShapeDtypeStruct((NPAD, _D8), jnp.float32)],
)


def _stage3_body(dis_ref, zp_ref, y3_ref, b3_ref, wc_ref, bc_ref, out_ref, h3_ref):
    dis = dis_ref[...]
    h3 = jnp.tanh(
        dis * (zp_ref[0][:, :2] + zp_ref[1][:, :2] + y3_ref[:, :2]) + b3_ref[...]
    )
    h3_ref[...] = h3
    out_ref[...] = _mm(h3, wc_ref[...]) + bc_ref[...]


_stage3 = pl.pallas_call(
    _stage3_body,
    grid=(_GRID,),
    in_specs=[
        _vspec(1), _pspec(_D8), _vspec(_D8),
        _wspec((1, 2)), _wspec((2, 1)), _wspec((1, 1)),
    ],
    out_specs=[_vspec(1), _vspec(2)],
    out_shape=[
        jax.ShapeDtypeStruct((NPAD, 1), jnp.float32),
        jax.ShapeDtypeStruct((NPAD, 2), jnp.float32),
    ],
)


def kernel(x, edge_index, W1, b1, W2, b2, W3, b3, Wc, bc):
    padi = jnp.full((E_PAD - _E,), _N, dtype=jnp.int32)
    src = jnp.concatenate([edge_index[0], padi])
    dst = jnp.concatenate([edge_index[1], padi])
    xp = jnp.pad(x, ((0, NPAD - _N), (0, 0)))

    z1 = jnp.zeros((NPAD,), jnp.float32)
    z8 = jnp.zeros((NPAD, _D8), jnp.float32)
    ones = jnp.ones((CHUNK,), jnp.float32)

    degp = _deg(dst, z1, ones)
    dis, y1 = _stage_dis(degp.reshape(_NC, NPAD, 1), xp)
    z1p = _prop1(src, dst, y1.reshape(NPAD), z1)
    (y2,) = _stage1(
        dis, z1p.reshape(_NC, NPAD, 1), y1, W1.reshape(1, 4), b1.reshape(1, 4)
    )
    z2p = _prop8(src, dst, y2, z8)
    (y3,) = _stage2(dis, z2p, y2, W2, b2.reshape(1, 4), W3)
    z3p = _prop8(src, dst, y3, z8)
    out, h3 = _stage3(dis, z3p, y3, b3.reshape(1, 2), Wc, bc.reshape(1, 1))
    return (out[:_N], h3[:_N])


# speedup vs baseline: 97.3003x; 1.1018x over previous
import functools

import jax
import jax.numpy as jnp
from jax import lax
from jax.experimental import pallas as pl
from jax.experimental.pallas import tpu as pltpu
from jax.experimental.pallas import tpu_sc as plsc

_N = 100000
_E = 6400000
_NC = 2
_NS = 16
_NW = _NC * _NS
_D8 = 8

CHUNK = 128
GRP = 8
CPT = 1568
GROUPS = CPT // GRP
E_PAD = _NW * CPT * CHUNK
NPAD = 100352
RPT = NPAD // _NS

_mesh = plsc.VectorSubcoreMesh(
    core_axis_name="c", subcore_axis_name="s", num_cores=_NC, num_subcores=_NS
)
_cparams = pltpu.CompilerParams(use_tc_tiling_on_sc=False)


def _make_deg():
    @functools.partial(
        pl.kernel,
        out_type=jax.ShapeDtypeStruct((_NC, NPAD), jnp.float32),
        mesh=_mesh,
        compiler_params=_cparams,
        scratch_types=[
            pltpu.VMEM_SHARED((NPAD,), jnp.float32),
            pltpu.VMEM((CHUNK,), jnp.float32),
            [[pltpu.VMEM((CHUNK,), jnp.int32) for _ in range(GRP)]
             for _ in range(2)],
            pltpu.SemaphoreType.DMA,
            pltpu.SemaphoreType.DMA,
        ],
    )
    def deg_kernel(dst_hbm, zz_hbm, ones_hbm, out_hbm, z_sp, ones_v, idxd, sem_i, sem_s):
        c = lax.axis_index("c")
        s = lax.axis_index("s")
        r0 = s * RPT
        pltpu.sync_copy(zz_hbm.at[pl.ds(r0, RPT)], z_sp.at[pl.ds(r0, RPT)])
        pltpu.sync_copy(ones_hbm, ones_v)
        plsc.subcore_barrier()
        base = (c * _NS + s) * CPT * CHUNK

        def stage(g, p):
            e0 = base + g * (GRP * CHUNK)
            for j in range(GRP):
                pltpu.async_copy(
                    dst_hbm.at[pl.ds(e0 + j * CHUNK, CHUNK)], idxd[p][j], sem_i
                )

        def wait_idx(p):
            for j in range(GRP):
                pltpu.make_async_copy(
                    dst_hbm.at[pl.ds(0, CHUNK)], idxd[p][j], sem_i
                ).wait()

        stage(0, 0)

        def scat(p):
            return [
                pltpu.async_copy(ones_v, z_sp.at[idxd[p][j]], sem_s, add=True)
                for j in range(GRP)
            ]

        def pair(h, carry):
            g = h * 2
            wait_idx(0)
            stage(g + 1, 1)
            s0 = scat(0)
            wait_idx(1)
            for d in s0:
                d.wait()
            @pl.when(g + 2 < GROUPS)
            def _():
                stage(g + 2, 0)
            s1 = scat(1)
            for d in s1:
                d.wait()
            return carry

        lax.fori_loop(0, GROUPS // 2, pair, 0)
        plsc.subcore_barrier()
        pltpu.sync_copy(z_sp.at[pl.ds(r0, RPT)], out_hbm.at[c, pl.ds(r0, RPT)])

    return deg_kernel


def _make_prop(D):
    one_d = D == 1
    tshape = (NPAD,) if one_d else (NPAD, D)
    rshape = (CHUNK,) if one_d else (CHUNK, D)

    pgrp = 4
    pgroups = CPT // pgrp

    @functools.partial(
        pl.kernel,
        out_type=jax.ShapeDtypeStruct((_NC,) + tshape, jnp.float32),
        mesh=_mesh,
        compiler_params=_cparams,
        scratch_types=[
            pltpu.VMEM_SHARED(tshape, jnp.float32),
            pltpu.VMEM_SHARED(tshape, jnp.float32),
            [pltpu.VMEM((pgrp * CHUNK,), jnp.int32) for _ in range(2)],
            [[pltpu.VMEM((CHUNK,), jnp.int32) for _ in range(pgrp)]
             for _ in range(2)],
            [pltpu.VMEM(rshape, jnp.float32) for _ in range(pgrp)],
            pltpu.SemaphoreType.DMA,
            pltpu.SemaphoreType.DMA,
            pltpu.SemaphoreType.DMA,
        ],
    )
    def prop_kernel(
        src_hbm, dst_hbm, y_hbm, zz_hbm, out_hbm,
        y_sp, z_sp, idxs, idxd, rows, sem_i, sem_g, sem_s,
    ):
        c = lax.axis_index("c")
        s = lax.axis_index("s")
        r0 = s * RPT
        pltpu.sync_copy(y_hbm.at[pl.ds(r0, RPT)], y_sp.at[pl.ds(r0, RPT)])
        pltpu.sync_copy(zz_hbm.at[pl.ds(r0, RPT)], z_sp.at[pl.ds(r0, RPT)])
        plsc.subcore_barrier()
        base = (c * _NS + s) * CPT * CHUNK

        def stage(g, p):
            e0 = base + g * (pgrp * CHUNK)
            for j in range(pgrp):
                pltpu.async_copy(
                    dst_hbm.at[pl.ds(e0 + j * CHUNK, CHUNK)], idxd[p][j], sem_i
                )
            pltpu.async_copy(src_hbm.at[pl.ds(e0, pgrp * CHUNK)], idxs[p], sem_i)

        def wait_idx(p):
            for j in range(pgrp):
                pltpu.make_async_copy(
                    dst_hbm.at[pl.ds(0, CHUNK)], idxd[p][j], sem_i
                ).wait()
            pltpu.make_async_copy(
                src_hbm.at[pl.ds(0, pgrp * CHUNK)], idxs[p], sem_i
            ).wait()

        def run_group(p):
            gs = [
                pltpu.async_copy(
                    y_sp.at[idxs[p].at[pl.ds(j * CHUNK, CHUNK)]], rows[j], sem_g
                )
                for j in range(pgrp)
            ]
            ss = []
            for j in range(pgrp):
                gs[j].wait()
                ss.append(
                    pltpu.async_copy(rows[j], z_sp.at[idxd[p][j]], sem_s, add=True)
                )
            return ss

        stage(0, 0)

        def pair(h, carry):
            g = h * 2
            wait_idx(0)
            stage(g + 1, 1)
            s0 = run_group(0)
            wait_idx(1)
            for d in s0:
                d.wait()
            s1 = run_group(1)
            @pl.when(g + 2 < pgroups)
            def _():
                stage(g + 2, 0)
            for d in s1:
                d.wait()
            return carry

        lax.fori_loop(0, pgroups // 2, pair, 0)
        plsc.subcore_barrier()
        pltpu.sync_copy(z_sp.at[pl.ds(r0, RPT)], out_hbm.at[c, pl.ds(r0, RPT)])

    return prop_kernel


_deg = _make_deg()
_prop1 = _make_prop(1)
_prop8 = _make_prop(_D8)


_BLK = 2048
_GRID = NPAD // _BLK


def _vspec(c):
    return pl.BlockSpec((_BLK, c), lambda i: (i, 0))


def _pspec(c):
    return pl.BlockSpec((_NC, _BLK, c), lambda i: (0, i, 0))


def _wspec(shape):
    return pl.BlockSpec(shape, lambda i: tuple(0 for _ in shape))


def _mm(a, w):
    k = w.shape[0]
    acc = a[:, 0:1] * w[0:1, :]
    for i in range(1, k):
        acc = acc + a[:, i : i + 1] * w[i : i + 1, :]
    return acc


def _pad8(a):
    return jnp.pad(a, ((0, 0), (0, _D8 - a.shape[1])))


def _mmbf(a, w):
    ab = a.astype(jnp.bfloat16).astype(jnp.float32)
    wb = w.astype(jnp.bfloat16).astype(jnp.float32)
    return _mm(ab, wb)


def _stage_dis_body(degp_ref, x_ref, dis_ref, y1_ref):
    deg = degp_ref[0] + degp_ref[1] + 1.0
    dis = lax.rsqrt(deg)
    dis_ref[...] = dis
    y1_ref[...] = dis * x_ref[...]


_stage_dis = pl.pallas_call(
    _stage_dis_body,
    grid=(_GRID,),
    in_specs=[_pspec(1), _vspec(1)],
    out_specs=[_vspec(1), _vspec(1)],
    out_shape=[
        jax.ShapeDtypeStruct((NPAD, 1), jnp.float32),
        jax.ShapeDtypeStruct((NPAD, 1), jnp.float32),
    ],
)


def _stage1_body(dis_ref, zp_ref, y1_ref, w1_ref, b1_ref, w2_ref, y2_ref):
    dis = dis_ref[...]
    t = dis * (zp_ref[0] + zp_ref[1] + y1_ref[...])
    h1 = jnp.tanh(t * w1_ref[...] + b1_ref[...])
    hw2 = _mmbf(h1, w2_ref[...])
    y2_ref[...] = _pad8(dis * hw2)


_stage1 = pl.pallas_call(
    _stage1_body,
    grid=(_GRID,),
    in_specs=[
        _vspec(1), _pspec(1), _vspec(1),
        _wspec((1, 4)), _wspec((1, 4)), _wspec((4, 4)),
    ],
    out_specs=[_vspec(_D8)],
    out_shape=[jax.ShapeDtypeStruct((NPAD, _D8), jnp.float32)],
)


def _stage2_body(dis_ref, zp_ref, y2_ref, b2_ref, w3_ref, y3_ref):
    dis = dis_ref[...]
    h2 = jnp.tanh(
        dis * (zp_ref[0][:, :4] + zp_ref[1][:, :4] + y2_ref[:, :4]) + b2_ref[...]
    )
    y3_ref[...] = _pad8(dis * _mmbf(h2, w3_ref[...]))


_stage2 = pl.pallas_call(
    _stage2_body,
    grid=(_GRID,),
    in_specs=[
        _vspec(1), _pspec(_D8), _vspec(_D8),
        _wspec((1, 4)), _wspec((4, 2)),
    ],
    out_specs=[_vspec(_D8)],
    out_shape=[jax.ShapeDtypeStruct((NPAD, _D8), jnp.float32)],
)


def _stage3_body(dis_ref, zp_ref, y3_ref, b3_ref, wc_ref, bc_ref, out_ref, h3_ref):
    dis = dis_ref[...]
    h3 = jnp.tanh(
        dis * (zp_ref[0][:, :2] + zp_ref[1][:, :2] + y3_ref[:, :2]) + b3_ref[...]
    )
    h3_ref[...] = h3
    out_ref[...] = _mmbf(h3, wc_ref[...]) + bc_ref[...]


_stage3 = pl.pallas_call(
    _stage3_body,
    grid=(_GRID,),
    in_specs=[
        _vspec(1), _pspec(_D8), _vspec(_D8),
        _wspec((1, 2)), _wspec((2, 1)), _wspec((1, 1)),
    ],
    out_specs=[_vspec(1), _vspec(2)],
    out_shape=[
        jax.ShapeDtypeStruct((NPAD, 1), jnp.float32),
        jax.ShapeDtypeStruct((NPAD, 2), jnp.float32),
    ],
)


def kernel(x, edge_index, W1, b1, W2, b2, W3, b3, Wc, bc):
    padi = jnp.full((E_PAD - _E,), _N, dtype=jnp.int32)
    src = jnp.concatenate([edge_index[0], padi])
    dst = jnp.concatenate([edge_index[1], padi])
    xp = jnp.pad(x, ((0, NPAD - _N), (0, 0)))

    z1 = jnp.zeros((NPAD,), jnp.float32)
    z8 = jnp.zeros((NPAD, _D8), jnp.float32)
    ones = jnp.ones((CHUNK,), jnp.float32)

    degp = _deg(dst, z1, ones)
    dis, y1 = _stage_dis(degp.reshape(_NC, NPAD, 1), xp)
    z1p = _prop1(src, dst, y1.reshape(NPAD), z1)
    (y2,) = _stage1(
        dis, z1p.reshape(_NC, NPAD, 1), y1, W1.reshape(1, 4), b1.reshape(1, 4), W2
    )
    z2p = _prop8(src, dst, y2, z8)
    (y3,) = _stage2(dis, z2p, y2, b2.reshape(1, 4), W3)
    z3p = _prop8(src, dst, y3, z8)
    out, h3 = _stage3(dis, z3p, y3, b3.reshape(1, 2), Wc, bc.reshape(1, 1))
    return (out[:_N], h3[:_N])
